# baseline (speedup 1.0000x reference)
import functools

import jax
import jax.numpy as jnp
from jax.experimental import pallas as pl

B = 16384
INPUT_DIM = 512
EMB_DIM = 128
N_CLUSTERS = 64
BLOCK_ROWS = 2048
NB = B // BLOCK_ROWS


def _fused_body(x_ref, pca_ref, cent_ref, out_ref):
    xb = x_ref[...]
    pe = pca_ref[...]

    x_enc_t = jax.lax.dot_general(
        pe, xb, (((1,), (1,)), ((), ())),
        preferred_element_type=jnp.float32)

    cen = cent_ref[...]
    cen_p = jnp.concatenate(
        [cen, jnp.zeros((128 - N_CLUSTERS, EMB_DIM), jnp.float32)], axis=0)

    cross_t = jax.lax.dot_general(
        cen_p, x_enc_t, (((1,), (0,)), ((), ())),
        preferred_element_type=jnp.float32)

    x_sq = x_enc_t * x_enc_t
    x2row = jax.lax.dot_general(
        jnp.ones((8, EMB_DIM), jnp.float32), x_sq, (((1,), (0,)), ((), ())),
        preferred_element_type=jnp.float32)[:1]

    c2 = jnp.sum(cen_p * cen_p, axis=1, keepdims=True)
    pad = jax.lax.broadcasted_iota(jnp.int32, (128, 1), 0) >= N_CLUSTERS
    c2 = jnp.where(pad, jnp.float32(3e38), c2)
    m = jnp.min(c2 - 2.0 * cross_t, axis=0, keepdims=True)
    out_ref[...] = jnp.sqrt(jnp.maximum(x2row + m, 0.0))[None]


@functools.partial(jax.jit, static_argnames=("interpret",))
def kernel(x, pca_components, centroids, interpret=False):
    return pl.pallas_call(
        _fused_body,
        grid=(NB,),
        in_specs=[
            pl.BlockSpec((BLOCK_ROWS, INPUT_DIM), lambda i: (i, 0)),
            pl.BlockSpec((EMB_DIM, INPUT_DIM), lambda i: (0, 0)),
            pl.BlockSpec((N_CLUSTERS, EMB_DIM), lambda i: (0, 0)),
        ],
        out_specs=pl.BlockSpec((1, 1, BLOCK_ROWS), lambda i: (i, 0, 0)),
        out_shape=jax.ShapeDtypeStruct((NB, 1, BLOCK_ROWS), jnp.float32),
        interpret=interpret,
    )(x, pca_components, centroids).reshape(B)


# speedup vs baseline: 1.0263x; 1.0263x over previous
import functools

import jax
import jax.numpy as jnp
from jax.experimental import pallas as pl

B = 16384
INPUT_DIM = 512
EMB_DIM = 128
N_CLUSTERS = 64
BLOCK_ROWS = 8192
NB = B // BLOCK_ROWS


def _fused_body(x_ref, pca_ref, cent_ref, out_ref):
    xb = x_ref[...]
    pe = pca_ref[...]

    x_enc_t = jax.lax.dot_general(
        pe, xb, (((1,), (1,)), ((), ())),
        preferred_element_type=jnp.float32)

    cen = cent_ref[...]
    cen_p = jnp.concatenate(
        [cen, jnp.zeros((128 - N_CLUSTERS, EMB_DIM), jnp.float32)], axis=0)

    cross_t = jax.lax.dot_general(
        cen_p, x_enc_t, (((1,), (0,)), ((), ())),
        preferred_element_type=jnp.float32)

    x_sq = x_enc_t * x_enc_t
    x2row = jax.lax.dot_general(
        jnp.ones((8, EMB_DIM), jnp.float32), x_sq, (((1,), (0,)), ((), ())),
        preferred_element_type=jnp.float32)[:1]

    c2 = jnp.sum(cen_p * cen_p, axis=1, keepdims=True)
    pad = jax.lax.broadcasted_iota(jnp.int32, (128, 1), 0) >= N_CLUSTERS
    c2 = jnp.where(pad, jnp.float32(3e38), c2)
    m = jnp.min(c2 - 2.0 * cross_t, axis=0, keepdims=True)
    out_ref[...] = jnp.sqrt(jnp.maximum(x2row + m, 0.0))[None]


@functools.partial(jax.jit, static_argnames=("interpret",))
def kernel(x, pca_components, centroids, interpret=False):
    return pl.pallas_call(
        _fused_body,
        grid=(NB,),
        in_specs=[
            pl.BlockSpec((BLOCK_ROWS, INPUT_DIM), lambda i: (i, 0)),
            pl.BlockSpec((EMB_DIM, INPUT_DIM), lambda i: (0, 0)),
            pl.BlockSpec((N_CLUSTERS, EMB_DIM), lambda i: (0, 0)),
        ],
        out_specs=pl.BlockSpec((1, 1, BLOCK_ROWS), lambda i: (i, 0, 0)),
        out_shape=jax.ShapeDtypeStruct((NB, 1, BLOCK_ROWS), jnp.float32),
        interpret=interpret,
    )(x, pca_components, centroids).reshape(B)


# speedup vs baseline: 1.0885x; 1.0606x over previous
import functools

import jax
import jax.numpy as jnp
from jax.experimental import pallas as pl

B = 16384
INPUT_DIM = 512
EMB_DIM = 128
N_CLUSTERS = 64
BLOCK_ROWS = 4096
NB = B // BLOCK_ROWS


def _fused_body(x1_ref, x2_ref, pca_ref, cent_ref, out_ref):
    pe = pca_ref[...]

    HALF = INPUT_DIM // 2
    x_enc_t = jax.lax.dot_general(
        pe[:, :HALF], x1_ref[...], (((1,), (1,)), ((), ())),
        preferred_element_type=jnp.float32)
    x_enc_t = x_enc_t + jax.lax.dot_general(
        pe[:, HALF:], x2_ref[...], (((1,), (1,)), ((), ())),
        preferred_element_type=jnp.float32)

    cen = cent_ref[...]
    cen_p = jnp.concatenate(
        [cen, jnp.zeros((128 - N_CLUSTERS, EMB_DIM), jnp.float32)], axis=0)

    cross_t = jax.lax.dot_general(
        cen_p, x_enc_t, (((1,), (0,)), ((), ())),
        preferred_element_type=jnp.float32)

    x_sq = x_enc_t * x_enc_t
    x2row = jax.lax.dot_general(
        jnp.ones((8, EMB_DIM), jnp.float32), x_sq, (((1,), (0,)), ((), ())),
        preferred_element_type=jnp.float32)[:1]

    c2 = jnp.sum(cen_p * cen_p, axis=1, keepdims=True)
    pad = jax.lax.broadcasted_iota(jnp.int32, (128, 1), 0) >= N_CLUSTERS
    c2 = jnp.where(pad, jnp.float32(3e38), c2)
    m = jnp.min(c2 - 2.0 * cross_t, axis=0, keepdims=True)
    out_ref[...] = jnp.sqrt(jnp.maximum(x2row + m, 0.0))[None]


@functools.partial(jax.jit, static_argnames=("interpret",))
def kernel(x, pca_components, centroids, interpret=False):
    return pl.pallas_call(
        _fused_body,
        grid=(NB,),
        in_specs=[
            pl.BlockSpec((BLOCK_ROWS, INPUT_DIM // 2), lambda i: (i, 0)),
            pl.BlockSpec((BLOCK_ROWS, INPUT_DIM // 2), lambda i: (i, 1)),
            pl.BlockSpec((EMB_DIM, INPUT_DIM), lambda i: (0, 0)),
            pl.BlockSpec((N_CLUSTERS, EMB_DIM), lambda i: (0, 0)),
        ],
        out_specs=pl.BlockSpec((1, 1, BLOCK_ROWS), lambda i: (i, 0, 0)),
        out_shape=jax.ShapeDtypeStruct((NB, 1, BLOCK_ROWS), jnp.float32),
        interpret=interpret,
    )(x, x, pca_components, centroids).reshape(B)
